# Initial kernel scaffold; baseline (speedup 1.0000x reference)
import jax
import jax.numpy as jnp
from jax.experimental import pallas as pl


def kernel(x, target, mask, m_list):
    raise NotImplementedError("write your pallas kernel here")


# speedup vs baseline: 2.8049x; 2.8049x over previous
import functools

import jax
import jax.numpy as jnp
from jax import lax
from jax.experimental import pallas as pl
from jax.experimental.pallas import tpu as pltpu
from jax.experimental.pallas import tpu_sc as plsc

NC = 2
NS = 16
L = 16
NW = NC * NS

B = 16384
C = 128
ROWS_PER_TILE = B // NW
GROUPS = ROWS_PER_TILE // L

_LN2 = 0.6931471805599453
_SQRT2 = 1.4142135623730951


def _log16(s):
    bits = lax.bitcast_convert_type(s, jnp.int32)
    e = lax.shift_right_logical(bits, 23) - 127
    mant = lax.bitwise_or(lax.bitwise_and(bits, 0x7FFFFF), 0x3F800000)
    m = lax.bitcast_convert_type(mant, jnp.float32)
    big = m > _SQRT2
    m = jnp.where(big, m * 0.5, m)
    ef = e.astype(jnp.float32) + jnp.where(big, 1.0, 0.0)
    t = (m - 1.0) / (m + 1.0)
    t2 = t * t
    poly = t * (2.0 + t2 * (2.0 / 3.0 + t2 * (2.0 / 5.0 + t2 * (2.0 / 7.0))))
    return ef * _LN2 + poly


def _tile_body(x_hbm, tgt_hbm, maskf_hbm, mlist_hbm, out_hbm,
               x_v, tgt_v, maskf_v, mlist_v, res_v):
    wid = lax.axis_index("s") * NC + lax.axis_index("c")
    rbase = wid * ROWS_PER_TILE

    pltpu.sync_copy(x_hbm.at[pl.ds(rbase * C, ROWS_PER_TILE * C)], x_v)
    pltpu.sync_copy(tgt_hbm.at[pl.ds(rbase, ROWS_PER_TILE)], tgt_v)
    pltpu.sync_copy(maskf_hbm.at[pl.ds(rbase, ROWS_PER_TILE)], maskf_v)
    pltpu.sync_copy(mlist_hbm, mlist_v)

    lane = lax.iota(jnp.int32, L)

    def group_body(g, carry):
        acc, cnt = carry
        rows = g * L + lane
        fbase = rows * C
        mx = plsc.load_gather(x_v, [fbase])
        for j in range(1, C):
            mx = jnp.maximum(mx, plsc.load_gather(x_v, [fbase + j]))
        s = jnp.zeros((L,), jnp.float32)
        for j in range(C):
            s = s + jnp.exp(plsc.load_gather(x_v, [fbase + j]) - mx)
        t = plsc.load_gather(tgt_v, [rows])
        mk = plsc.load_gather(maskf_v, [rows])
        xt = plsc.load_gather(x_v, [fbase + t])
        mr = plsc.load_gather(mlist_v, [t])
        s = s - jnp.exp(xt - mx) + jnp.exp(xt - mr - mx)
        loss = mx + _log16(s) - xt + mr
        return acc + loss * mk, cnt + mk

    zero = jnp.zeros((L,), jnp.float32)
    acc, cnt = lax.fori_loop(0, GROUPS, group_body, (zero, zero))

    acc_s = jnp.sum(acc)
    cnt_s = jnp.sum(cnt)
    res = jnp.where(lane == 0, acc_s, jnp.where(lane == 1, cnt_s, 0.0))
    res_v[...] = res
    pltpu.sync_copy(res_v, out_hbm.at[pl.ds(wid * L, L)])


@jax.jit
def _ldam_partials(x1d, target, maskf, m_list):
    mesh = plsc.VectorSubcoreMesh(
        core_axis_name="c", subcore_axis_name="s",
        num_cores=NC, num_subcores=NS)
    return pl.kernel(
        _tile_body,
        out_type=jax.ShapeDtypeStruct((NW * L,), jnp.float32),
        mesh=mesh,
        compiler_params=pltpu.CompilerParams(needs_layout_passes=False),
        scratch_types=[
            pltpu.VMEM((ROWS_PER_TILE * C,), jnp.float32),
            pltpu.VMEM((ROWS_PER_TILE,), jnp.int32),
            pltpu.VMEM((ROWS_PER_TILE,), jnp.float32),
            pltpu.VMEM((C,), jnp.float32),
            pltpu.VMEM((L,), jnp.float32),
        ],
    )(x1d, target, maskf, m_list)


def kernel(x, target, mask, m_list):
    x1d = x.reshape(-1)
    target = target.reshape(-1).astype(jnp.int32)
    maskf = mask.reshape(-1).astype(jnp.float32)
    partials = _ldam_partials(x1d, target, maskf, m_list).reshape(NW, L)
    return jnp.sum(partials[:, 0]) / jnp.sum(partials[:, 1])


# speedup vs baseline: 4.0275x; 1.4359x over previous
import functools

import jax
import jax.numpy as jnp
from jax import lax
from jax.experimental import pallas as pl
from jax.experimental.pallas import tpu as pltpu
from jax.experimental.pallas import tpu_sc as plsc

NC = 2
NS = 16
L = 16
NW = NC * NS

B = 16384
C = 128
ROWS_PER_TILE = B // NW
GROUPS = ROWS_PER_TILE // L

_LN2 = 0.6931471805599453
_SQRT2 = 1.4142135623730951


def _log16(s):
    bits = lax.bitcast_convert_type(s, jnp.int32)
    e = lax.shift_right_logical(bits, 23) - 127
    mant = lax.bitwise_or(lax.bitwise_and(bits, 0x7FFFFF), 0x3F800000)
    m = lax.bitcast_convert_type(mant, jnp.float32)
    big = m > _SQRT2
    m = jnp.where(big, m * 0.5, m)
    ef = e.astype(jnp.float32) + jnp.where(big, 1.0, 0.0)
    t = (m - 1.0) / (m + 1.0)
    t2 = t * t
    poly = t * (2.0 + t2 * (2.0 / 3.0 + t2 * (2.0 / 5.0 + t2 * (2.0 / 7.0))))
    return ef * _LN2 + poly


def _tile_body(x_hbm, tgt_hbm, maskf_hbm, mlist_hbm, out_hbm,
               x_v, tgt_v, maskf_v, mlist_v, res_v):
    wid = lax.axis_index("s") * NC + lax.axis_index("c")
    rbase = wid * ROWS_PER_TILE

    pltpu.sync_copy(x_hbm.at[pl.ds(rbase * C, ROWS_PER_TILE * C)], x_v)
    pltpu.sync_copy(tgt_hbm.at[pl.ds(rbase, ROWS_PER_TILE)], tgt_v)
    pltpu.sync_copy(maskf_hbm.at[pl.ds(rbase, ROWS_PER_TILE)], maskf_v)
    pltpu.sync_copy(mlist_hbm, mlist_v)

    lane = lax.iota(jnp.int32, L)

    def group_body(g, carry):
        acc, cnt = carry
        rows = g * L + lane
        fbase = rows * C
        mx = plsc.load_gather(x_v, [fbase + lane])
        for j in range(1, C):
            mx = jnp.maximum(
                mx, plsc.load_gather(x_v, [fbase + ((j + lane) & (C - 1))]))
        s = jnp.zeros((L,), jnp.float32)
        for j in range(C):
            s = s + jnp.exp(
                plsc.load_gather(x_v, [fbase + ((j + lane) & (C - 1))]) - mx)
        t = plsc.load_gather(tgt_v, [rows])
        mk = plsc.load_gather(maskf_v, [rows])
        xt = plsc.load_gather(x_v, [fbase + t])
        mr = plsc.load_gather(mlist_v, [t])
        s = s - jnp.exp(xt - mx) + jnp.exp(xt - mr - mx)
        loss = mx + _log16(s) - xt + mr
        return acc + loss * mk, cnt + mk

    zero = jnp.zeros((L,), jnp.float32)
    acc, cnt = lax.fori_loop(0, GROUPS, group_body, (zero, zero))

    acc_s = jnp.sum(acc)
    cnt_s = jnp.sum(cnt)
    res = jnp.where(lane == 0, acc_s, jnp.where(lane == 1, cnt_s, 0.0))
    res_v[...] = res
    pltpu.sync_copy(res_v, out_hbm.at[pl.ds(wid * L, L)])


@jax.jit
def _ldam_partials(x1d, target, maskf, m_list):
    mesh = plsc.VectorSubcoreMesh(
        core_axis_name="c", subcore_axis_name="s",
        num_cores=NC, num_subcores=NS)
    return pl.kernel(
        _tile_body,
        out_type=jax.ShapeDtypeStruct((NW * L,), jnp.float32),
        mesh=mesh,
        compiler_params=pltpu.CompilerParams(needs_layout_passes=False),
        scratch_types=[
            pltpu.VMEM((ROWS_PER_TILE * C,), jnp.float32),
            pltpu.VMEM((ROWS_PER_TILE,), jnp.int32),
            pltpu.VMEM((ROWS_PER_TILE,), jnp.float32),
            pltpu.VMEM((C,), jnp.float32),
            pltpu.VMEM((L,), jnp.float32),
        ],
    )(x1d, target, maskf, m_list)


def kernel(x, target, mask, m_list):
    x1d = x.reshape(-1)
    target = target.reshape(-1).astype(jnp.int32)
    maskf = mask.reshape(-1).astype(jnp.float32)
    partials = _ldam_partials(x1d, target, maskf, m_list).reshape(NW, L)
    return jnp.sum(partials[:, 0]) / jnp.sum(partials[:, 1])


# speedup vs baseline: 4.3795x; 1.0874x over previous
import functools

import jax
import jax.numpy as jnp
from jax import lax
from jax.experimental import pallas as pl
from jax.experimental.pallas import tpu as pltpu
from jax.experimental.pallas import tpu_sc as plsc

NC = 2
NS = 16
L = 16
NW = NC * NS

B = 16384
C = 128
ROWS_PER_TILE = B // NW
GROUPS = ROWS_PER_TILE // L

_LN2 = 0.6931471805599453
_SQRT2 = 1.4142135623730951


def _log16(s):
    bits = lax.bitcast_convert_type(s, jnp.int32)
    e = lax.shift_right_logical(bits, 23) - 127
    mant = lax.bitwise_or(lax.bitwise_and(bits, 0x7FFFFF), 0x3F800000)
    m = lax.bitcast_convert_type(mant, jnp.float32)
    big = m > _SQRT2
    m = jnp.where(big, m * 0.5, m)
    ef = e.astype(jnp.float32) + jnp.where(big, 1.0, 0.0)
    t = (m - 1.0) / (m + 1.0)
    t2 = t * t
    poly = t * (2.0 + t2 * (2.0 / 3.0 + t2 * (2.0 / 5.0 + t2 * (2.0 / 7.0))))
    return ef * _LN2 + poly


def _tile_body(x_hbm, tgt_hbm, maskf_hbm, mlist_hbm, out_hbm,
               x_v, tgt_v, maskf_v, mlist_v, res_v):
    wid = lax.axis_index("s") * NC + lax.axis_index("c")
    rbase = wid * ROWS_PER_TILE

    pltpu.sync_copy(x_hbm.at[pl.ds(rbase * C, ROWS_PER_TILE * C)], x_v)
    pltpu.sync_copy(tgt_hbm.at[pl.ds(rbase, ROWS_PER_TILE)], tgt_v)
    pltpu.sync_copy(maskf_hbm.at[pl.ds(rbase, ROWS_PER_TILE)], maskf_v)
    pltpu.sync_copy(mlist_hbm, mlist_v)

    lane = lax.iota(jnp.int32, L)

    def group_body(g, carry):
        acc, cnt = carry
        rows = g * L + lane
        fbase = rows * C
        mx = plsc.load_gather(x_v, [fbase + lane])
        for j in range(1, C):
            mx = jnp.maximum(
                mx, plsc.load_gather(x_v, [fbase + ((j + lane) & (C - 1))]))
        s = jnp.zeros((L,), jnp.float32)
        for j in range(C):
            s = s + jnp.exp(
                plsc.load_gather(x_v, [fbase + ((j - lane) & (C - 1))]) - mx)
        t = plsc.load_gather(tgt_v, [rows])
        mk = plsc.load_gather(maskf_v, [rows])
        xt = plsc.load_gather(x_v, [fbase + t])
        mr = plsc.load_gather(mlist_v, [t])
        s = s - jnp.exp(xt - mx) + jnp.exp(xt - mr - mx)
        loss = mx + _log16(s) - xt + mr
        return acc + loss * mk, cnt + mk

    zero = jnp.zeros((L,), jnp.float32)
    acc, cnt = lax.fori_loop(0, GROUPS, group_body, (zero, zero))

    acc_s = jnp.sum(acc)
    cnt_s = jnp.sum(cnt)
    res = jnp.where(lane == 0, acc_s, jnp.where(lane == 1, cnt_s, 0.0))
    res_v[...] = res
    pltpu.sync_copy(res_v, out_hbm.at[pl.ds(wid * L, L)])


@jax.jit
def _ldam_partials(x1d, target, maskf, m_list):
    mesh = plsc.VectorSubcoreMesh(
        core_axis_name="c", subcore_axis_name="s",
        num_cores=NC, num_subcores=NS)
    return pl.kernel(
        _tile_body,
        out_type=jax.ShapeDtypeStruct((NW * L,), jnp.float32),
        mesh=mesh,
        compiler_params=pltpu.CompilerParams(needs_layout_passes=False),
        scratch_types=[
            pltpu.VMEM((ROWS_PER_TILE * C,), jnp.float32),
            pltpu.VMEM((ROWS_PER_TILE,), jnp.int32),
            pltpu.VMEM((ROWS_PER_TILE,), jnp.float32),
            pltpu.VMEM((C,), jnp.float32),
            pltpu.VMEM((L,), jnp.float32),
        ],
    )(x1d, target, maskf, m_list)


def kernel(x, target, mask, m_list):
    x1d = x.reshape(-1)
    target = target.reshape(-1).astype(jnp.int32)
    maskf = mask.reshape(-1).astype(jnp.float32)
    partials = _ldam_partials(x1d, target, maskf, m_list).reshape(NW, L)
    return jnp.sum(partials[:, 0]) / jnp.sum(partials[:, 1])
